# Initial kernel scaffold; baseline (speedup 1.0000x reference)
import jax
import jax.numpy as jnp
from jax.experimental import pallas as pl


def kernel(hn, he, pos, fes, norm, edge_index, W_ev1, W_ev2, W_fc1, W_fc2, W_nu1, W_nu2):
    raise NotImplementedError("write your pallas kernel here")


# speedup vs baseline: 2.5268x; 2.5268x over previous
import functools

import jax
import jax.numpy as jnp
from jax import lax
from jax.experimental import pallas as pl
from jax.experimental.pallas import tpu as pltpu
from jax.experimental.pallas import tpu_sc as plsc

_NC = 2
_NS = 16
_NW = _NC * _NS
_CH = 80


def _tc_prep(hn, pos16, Wp1, Wp2):
    N, D = hn.shape
    FH = Wp1.shape[1]
    NB = 1000
    grid = (N // NB,)

    def body(hn_r, p_r, w1_r, w2_r, out_r):
        f32 = jnp.float32
        out_r[:, :D] = hn_r[...]
        out_r[:, D:D + FH] = jnp.dot(p_r[...], w1_r[...],
                                     preferred_element_type=f32)
        out_r[:, D + FH:] = jnp.dot(p_r[...], w2_r[...],
                                    preferred_element_type=f32)

    return pl.pallas_call(
        body,
        grid=grid,
        in_specs=[
            pl.BlockSpec((NB, D), lambda i: (i, 0)),
            pl.BlockSpec((NB, pos16.shape[1]), lambda i: (i, 0)),
            pl.BlockSpec(Wp1.shape, lambda i: (0, 0)),
            pl.BlockSpec(Wp2.shape, lambda i: (0, 0)),
        ],
        out_specs=pl.BlockSpec((NB, D + 2 * FH), lambda i: (i, 0)),
        out_shape=jax.ShapeDtypeStruct((N, D + 2 * FH), jnp.float32),
        compiler_params=pltpu.CompilerParams(
            dimension_semantics=("arbitrary",)),
    )(hn, pos16, Wp1, Wp2)


def _sc_gather(table, src, dst):
    N, TD = table.shape
    E = src.shape[0]
    per_w = E // _NW
    n_ch = per_w // _CH
    mesh = plsc.VectorSubcoreMesh(core_axis_name="c", subcore_axis_name="s",
                                  num_cores=_NC, num_subcores=_NS)

    @functools.partial(
        pl.kernel,
        out_type=(
            jax.ShapeDtypeStruct((E, TD), jnp.float32),
            jax.ShapeDtypeStruct((E, TD), jnp.float32),
        ),
        mesh=mesh,
        scratch_types=[
            pltpu.VMEM((_CH,), jnp.int32),
            pltpu.VMEM((_CH,), jnp.int32),
            pltpu.VMEM((_CH, TD), jnp.float32),
            pltpu.VMEM((_CH, TD), jnp.float32),
            pltpu.SemaphoreType.DMA,
        ],
    )
    def k(tab_hbm, src_hbm, dst_hbm, gs_hbm, gd_hbm,
          sidx, didx, srows, drows, sem):
        wid = lax.axis_index("s") * _NC + lax.axis_index("c")
        base = wid * per_w

        def body(j, carry):
            off = base + j * _CH
            pltpu.sync_copy(src_hbm.at[pl.ds(off, _CH)], sidx)
            pltpu.sync_copy(dst_hbm.at[pl.ds(off, _CH)], didx)
            c1 = pltpu.async_copy(tab_hbm.at[sidx], srows, sem)
            c2 = pltpu.async_copy(tab_hbm.at[didx], drows, sem)
            c1.wait()
            c2.wait()
            pltpu.sync_copy(srows, gs_hbm.at[pl.ds(off, _CH)])
            pltpu.sync_copy(drows, gd_hbm.at[pl.ds(off, _CH)])
            return carry

        lax.fori_loop(0, n_ch, body, 0)

    return k(table, src, dst)


def _sc_scatter(henw, dst, zeros_nd):
    E, D = henw.shape
    N = zeros_nd.shape[0]
    per_w = E // _NW
    n_ch = per_w // _CH
    n_cp = 10
    rows_pt = N // n_cp
    mesh = plsc.VectorSubcoreMesh(core_axis_name="c", subcore_axis_name="s",
                                  num_cores=_NC, num_subcores=_NS)

    @functools.partial(
        pl.kernel,
        out_type=jax.ShapeDtypeStruct((_NC, N, D), jnp.float32),
        mesh=mesh,
        scratch_types=[
            pltpu.VMEM((_CH,), jnp.int32),
            pltpu.VMEM((_CH, D), jnp.float32),
            pltpu.VMEM_SHARED((N, D), jnp.float32),
        ],
    )
    def k(henw_hbm, dst_hbm, zeros_hbm, out_hbm, idx, rows, acc):
        c = lax.axis_index("c")
        s = lax.axis_index("s")
        wid = s * _NC + c

        @pl.when(s == 0)
        def _():
            pltpu.sync_copy(zeros_hbm, acc)

        plsc.subcore_barrier()
        base = wid * per_w

        def body(j, carry):
            off = base + j * _CH
            pltpu.sync_copy(dst_hbm.at[pl.ds(off, _CH)], idx)
            pltpu.sync_copy(henw_hbm.at[pl.ds(off, _CH)], rows)
            pltpu.sync_copy(rows, acc.at[idx], add=True)
            return carry

        lax.fori_loop(0, n_ch, body, 0)
        plsc.subcore_barrier()

        @pl.when(s < n_cp)
        def _():
            pltpu.sync_copy(acc.at[pl.ds(s * rows_pt, rows_pt)],
                            out_hbm.at[c, pl.ds(s * rows_pt, rows_pt)])

    return k(henw, dst, zeros_nd)


def _tc_edge(he, gsrc, gdst, fes, norm2, W1e, W1s, W1d, Wev2, Wf, Wfc2):
    E, D = he.shape
    TD = gsrc.shape[1]
    FH = (TD - D) // 2
    NF = fes.shape[1]
    VD = Wev2.shape[1]
    EB = 512
    grid = (E // EB,)

    def body(he_r, gs_r, gd_r, fes_r, nrm_r,
             w1e_r, w1s_r, w1d_r, wev2_r, wf_r, wfc2_r,
             hen_r, henw_r):
        f32 = jnp.float32
        gs = gs_r[...]
        gd = gd_r[...]
        h1 = jnp.maximum(
            jnp.dot(he_r[...], w1e_r[...], preferred_element_type=f32)
            + jnp.dot(gs[:, :D], w1s_r[...], preferred_element_type=f32)
            + jnp.dot(gd[:, :D], w1d_r[...], preferred_element_type=f32),
            0.0)
        v = jnp.dot(h1, wev2_r[...], preferred_element_type=f32)
        r = jnp.maximum(
            gs[:, D:D + FH] + gd[:, D + FH:]
            + jnp.dot(fes_r[...], wf_r[...], preferred_element_type=f32),
            0.0)
        m = jnp.dot(r, wfc2_r[...], preferred_element_type=f32)
        tp = v[:, 0:1] * m[:, 0:D]
        for q in range(1, VD):
            tp = tp + v[:, q:q + 1] * m[:, q * D:(q + 1) * D]
        hen = he_r[...] + tp
        hen_r[...] = hen
        henw_r[...] = hen * nrm_r[...]

    blk = lambda rows, cols: pl.BlockSpec((rows, cols), lambda i: (i, 0))
    wblk = lambda shape: pl.BlockSpec(shape, lambda i: (0, 0))
    return pl.pallas_call(
        body,
        grid=grid,
        in_specs=[
            blk(EB, D), blk(EB, TD), blk(EB, TD), blk(EB, NF), blk(EB, 1),
            wblk(W1e.shape), wblk(W1s.shape), wblk(W1d.shape),
            wblk(Wev2.shape), wblk(Wf.shape), wblk(Wfc2.shape),
        ],
        out_specs=[blk(EB, D), blk(EB, D)],
        out_shape=[
            jax.ShapeDtypeStruct((E, D), jnp.float32),
            jax.ShapeDtypeStruct((E, D), jnp.float32),
        ],
        compiler_params=pltpu.CompilerParams(
            dimension_semantics=("arbitrary",)),
    )(he, gsrc, gdst, fes, norm2, W1e, W1s, W1d, Wev2, Wf, Wfc2)


def _tc_node(hn, parts, Wn1a, Wn1b, Wn2):
    N, D = hn.shape
    NB = 1000
    grid = (N // NB,)

    def body(hn_r, p_r, w1a_r, w1b_r, w2_r, out_r):
        f32 = jnp.float32
        t = p_r[0] + p_r[1]
        h = jnp.maximum(
            jnp.dot(hn_r[...], w1a_r[...], preferred_element_type=f32)
            + jnp.dot(t, w1b_r[...], preferred_element_type=f32),
            0.0)
        out_r[...] = hn_r[...] + jnp.dot(h, w2_r[...],
                                         preferred_element_type=f32)

    return pl.pallas_call(
        body,
        grid=grid,
        in_specs=[
            pl.BlockSpec((NB, D), lambda i: (i, 0)),
            pl.BlockSpec((2, NB, D), lambda i: (0, i, 0)),
            pl.BlockSpec(Wn1a.shape, lambda i: (0, 0)),
            pl.BlockSpec(Wn1b.shape, lambda i: (0, 0)),
            pl.BlockSpec(Wn2.shape, lambda i: (0, 0)),
        ],
        out_specs=pl.BlockSpec((NB, D), lambda i: (i, 0)),
        out_shape=jax.ShapeDtypeStruct((N, D), jnp.float32),
        compiler_params=pltpu.CompilerParams(
            dimension_semantics=("arbitrary",)),
    )(hn, parts, Wn1a, Wn1b, Wn2)


def kernel(hn, he, pos, fes, norm, edge_index,
           W_ev1, W_ev2, W_fc1, W_fc2, W_nu1, W_nu2):
    N, D = hn.shape
    src = edge_index[0]
    dst = edge_index[1]

    PD = 16
    pos16 = jnp.pad(pos, ((0, 0), (0, PD - pos.shape[1])))
    W1e, W1s, W1d = W_ev1[:D], W_ev1[D:2 * D], W_ev1[2 * D:]
    Wp1 = jnp.pad(W_fc1[0:3], ((0, PD - 3), (0, 0)))
    Wp2 = jnp.pad(W_fc1[3:6], ((0, PD - 3), (0, 0)))
    Wf = W_fc1[6:]
    Wn1a, Wn1b = W_nu1[:D], W_nu1[D:]
    norm2 = norm[:, None]
    zeros_nd = jnp.zeros((N, D), jnp.float32)

    table = _tc_prep(hn, pos16, Wp1, Wp2)
    gsrc, gdst = _sc_gather(table, src, dst)
    hen, henw = _tc_edge(he, gsrc, gdst, fes, norm2,
                         W1e, W1s, W1d, W_ev2, Wf, W_fc2)
    parts = _sc_scatter(henw, dst, zeros_nd)
    hnn = _tc_node(hn, parts, Wn1a, Wn1b, W_nu2)
    return (hnn, hen)


# speedup vs baseline: 2.5276x; 1.0003x over previous
import functools

import jax
import jax.numpy as jnp
from jax import lax
from jax.experimental import pallas as pl
from jax.experimental.pallas import tpu as pltpu
from jax.experimental.pallas import tpu_sc as plsc

_NC = 2
_NS = 16
_NW = _NC * _NS
_CH = 80


def _tc_prep(hn, pos16, Wp1, Wp2):
    N, D = hn.shape
    FH = Wp1.shape[1]
    NB = 1000
    grid = (N // NB,)

    def body(hn_r, p_r, w1_r, w2_r, out_r):
        f32 = jnp.float32
        out_r[:, :D] = hn_r[...]
        out_r[:, D:D + FH] = jnp.dot(p_r[...], w1_r[...],
                                     preferred_element_type=f32)
        out_r[:, D + FH:] = jnp.dot(p_r[...], w2_r[...],
                                    preferred_element_type=f32)

    return pl.pallas_call(
        body,
        grid=grid,
        in_specs=[
            pl.BlockSpec((NB, D), lambda i: (i, 0)),
            pl.BlockSpec((NB, pos16.shape[1]), lambda i: (i, 0)),
            pl.BlockSpec(Wp1.shape, lambda i: (0, 0)),
            pl.BlockSpec(Wp2.shape, lambda i: (0, 0)),
        ],
        out_specs=pl.BlockSpec((NB, D + 2 * FH), lambda i: (i, 0)),
        out_shape=jax.ShapeDtypeStruct((N, D + 2 * FH), jnp.float32),
        compiler_params=pltpu.CompilerParams(
            dimension_semantics=("arbitrary",)),
    )(hn, pos16, Wp1, Wp2)


def _sc_gather(table, src, dst):
    N, TD = table.shape
    E = src.shape[0]
    per_w = E // _NW
    n_ch = per_w // _CH
    mesh = plsc.VectorSubcoreMesh(core_axis_name="c", subcore_axis_name="s",
                                  num_cores=_NC, num_subcores=_NS)

    @functools.partial(
        pl.kernel,
        out_type=(
            jax.ShapeDtypeStruct((E, TD), jnp.float32),
            jax.ShapeDtypeStruct((E, TD), jnp.float32),
        ),
        mesh=mesh,
        scratch_types=[
            pltpu.VMEM((_CH,), jnp.int32),
            pltpu.VMEM((_CH,), jnp.int32),
            pltpu.VMEM((_CH, TD), jnp.float32),
            pltpu.VMEM((_CH, TD), jnp.float32),
            pltpu.SemaphoreType.DMA,
        ],
    )
    def k(tab_hbm, src_hbm, dst_hbm, gs_hbm, gd_hbm,
          sidx, didx, srows, drows, sem):
        wid = lax.axis_index("s") * _NC + lax.axis_index("c")
        base = wid * per_w

        def body(j, carry):
            off = base + j * _CH
            pltpu.sync_copy(src_hbm.at[pl.ds(off, _CH)], sidx)
            pltpu.sync_copy(dst_hbm.at[pl.ds(off, _CH)], didx)
            c1 = pltpu.async_copy(tab_hbm.at[sidx], srows, sem)
            c2 = pltpu.async_copy(tab_hbm.at[didx], drows, sem)
            c1.wait()
            c2.wait()
            pltpu.sync_copy(srows, gs_hbm.at[pl.ds(off, _CH)])
            pltpu.sync_copy(drows, gd_hbm.at[pl.ds(off, _CH)])
            return carry

        lax.fori_loop(0, n_ch, body, 0)

    return k(table, src, dst)


def _sc_scatter(henw, dst, zeros_nd):
    E, D = henw.shape
    N = zeros_nd.shape[0]
    per_w = E // _NW
    n_ch = per_w // _CH
    n_cp = 10
    rows_pt = N // n_cp
    mesh = plsc.VectorSubcoreMesh(core_axis_name="c", subcore_axis_name="s",
                                  num_cores=_NC, num_subcores=_NS)

    @functools.partial(
        pl.kernel,
        out_type=jax.ShapeDtypeStruct((_NC, N, D), jnp.float32),
        mesh=mesh,
        scratch_types=[
            pltpu.VMEM((_CH,), jnp.int32),
            pltpu.VMEM((_CH, D), jnp.float32),
            pltpu.VMEM_SHARED((N, D), jnp.float32),
        ],
    )
    def k(henw_hbm, dst_hbm, zeros_hbm, out_hbm, idx, rows, acc):
        c = lax.axis_index("c")
        s = lax.axis_index("s")
        wid = s * _NC + c

        @pl.when(s == 0)
        def _():
            pltpu.sync_copy(zeros_hbm, acc)

        plsc.subcore_barrier()
        base = wid * per_w

        def body(j, carry):
            off = base + j * _CH
            pltpu.sync_copy(dst_hbm.at[pl.ds(off, _CH)], idx)
            pltpu.sync_copy(henw_hbm.at[pl.ds(off, _CH)], rows)
            pltpu.sync_copy(rows, acc.at[idx], add=True)
            return carry

        lax.fori_loop(0, n_ch, body, 0)
        plsc.subcore_barrier()

        @pl.when(s < n_cp)
        def _():
            pltpu.sync_copy(acc.at[pl.ds(s * rows_pt, rows_pt)],
                            out_hbm.at[c, pl.ds(s * rows_pt, rows_pt)])

    return k(henw, dst, zeros_nd)


def _tc_edge(he, gsrc, gdst, fes, norm2, W1e, W1s, W1d, Wev2, Wf, Wfc2):
    E, D = he.shape
    TD = gsrc.shape[1]
    FH = (TD - D) // 2
    NF = fes.shape[1]
    VD = Wev2.shape[1]
    EB = 512
    grid = (E // EB,)

    def body(he_r, gs_r, gd_r, fes_r, nrm_r,
             w1e_r, w1s_r, w1d_r, wev2_r, wf_r, wfc2_r,
             hen_r, henw_r):
        f32 = jnp.float32
        b16 = jnp.bfloat16
        gs = gs_r[...]
        gd = gd_r[...]
        h1 = jnp.maximum(
            jnp.dot(he_r[...].astype(b16), w1e_r[...],
                    preferred_element_type=f32)
            + jnp.dot(gs[:, :D].astype(b16), w1s_r[...],
                      preferred_element_type=f32)
            + jnp.dot(gd[:, :D].astype(b16), w1d_r[...],
                      preferred_element_type=f32),
            0.0)
        v = jnp.dot(h1.astype(b16), wev2_r[...], preferred_element_type=f32)
        r = jnp.maximum(
            gs[:, D:D + FH] + gd[:, D + FH:]
            + jnp.dot(fes_r[...], wf_r[...], preferred_element_type=f32),
            0.0)
        m = jnp.dot(r.astype(b16), wfc2_r[...], preferred_element_type=f32)
        tp = v[:, 0:1] * m[:, 0:D]
        for q in range(1, VD):
            tp = tp + v[:, q:q + 1] * m[:, q * D:(q + 1) * D]
        hen = he_r[...] + tp
        hen_r[...] = hen
        henw_r[...] = hen * nrm_r[...]

    blk = lambda rows, cols: pl.BlockSpec((rows, cols), lambda i: (i, 0))
    wblk = lambda shape: pl.BlockSpec(shape, lambda i: (0, 0))
    return pl.pallas_call(
        body,
        grid=grid,
        in_specs=[
            blk(EB, D), blk(EB, TD), blk(EB, TD), blk(EB, NF), blk(EB, 1),
            wblk(W1e.shape), wblk(W1s.shape), wblk(W1d.shape),
            wblk(Wev2.shape), wblk(Wf.shape), wblk(Wfc2.shape),
        ],
        out_specs=[blk(EB, D), blk(EB, D)],
        out_shape=[
            jax.ShapeDtypeStruct((E, D), jnp.float32),
            jax.ShapeDtypeStruct((E, D), jnp.float32),
        ],
        compiler_params=pltpu.CompilerParams(
            dimension_semantics=("arbitrary",)),
    )(he, gsrc, gdst, fes, norm2, W1e, W1s, W1d, Wev2, Wf, Wfc2)


def _tc_node(hn, parts, Wn1a, Wn1b, Wn2):
    N, D = hn.shape
    NB = 1000
    grid = (N // NB,)

    def body(hn_r, p_r, w1a_r, w1b_r, w2_r, out_r):
        f32 = jnp.float32
        t = p_r[0] + p_r[1]
        h = jnp.maximum(
            jnp.dot(hn_r[...], w1a_r[...], preferred_element_type=f32)
            + jnp.dot(t, w1b_r[...], preferred_element_type=f32),
            0.0)
        out_r[...] = hn_r[...] + jnp.dot(h, w2_r[...],
                                         preferred_element_type=f32)

    return pl.pallas_call(
        body,
        grid=grid,
        in_specs=[
            pl.BlockSpec((NB, D), lambda i: (i, 0)),
            pl.BlockSpec((2, NB, D), lambda i: (0, i, 0)),
            pl.BlockSpec(Wn1a.shape, lambda i: (0, 0)),
            pl.BlockSpec(Wn1b.shape, lambda i: (0, 0)),
            pl.BlockSpec(Wn2.shape, lambda i: (0, 0)),
        ],
        out_specs=pl.BlockSpec((NB, D), lambda i: (i, 0)),
        out_shape=jax.ShapeDtypeStruct((N, D), jnp.float32),
        compiler_params=pltpu.CompilerParams(
            dimension_semantics=("arbitrary",)),
    )(hn, parts, Wn1a, Wn1b, Wn2)


def kernel(hn, he, pos, fes, norm, edge_index,
           W_ev1, W_ev2, W_fc1, W_fc2, W_nu1, W_nu2):
    N, D = hn.shape
    src = edge_index[0]
    dst = edge_index[1]

    PD = 16
    pos16 = jnp.pad(pos, ((0, 0), (0, PD - pos.shape[1])))
    W1e, W1s, W1d = W_ev1[:D], W_ev1[D:2 * D], W_ev1[2 * D:]
    Wp1 = jnp.pad(W_fc1[0:3], ((0, PD - 3), (0, 0)))
    Wp2 = jnp.pad(W_fc1[3:6], ((0, PD - 3), (0, 0)))
    Wf = W_fc1[6:]
    Wn1a, Wn1b = W_nu1[:D], W_nu1[D:]
    norm2 = norm[:, None]
    zeros_nd = jnp.zeros((N, D), jnp.float32)

    b16 = jnp.bfloat16
    table = _tc_prep(hn, pos16, Wp1, Wp2)
    gsrc, gdst = _sc_gather(table, src, dst)
    hen, henw = _tc_edge(he, gsrc, gdst, fes, norm2,
                         W1e.astype(b16), W1s.astype(b16), W1d.astype(b16),
                         W_ev2.astype(b16), Wf, W_fc2.astype(b16))
    parts = _sc_scatter(henw, dst, zeros_nd)
    hnn = _tc_node(hn, parts, Wn1a, Wn1b, W_nu2)
    return (hnn, hen)


# speedup vs baseline: 3.5338x; 1.3981x over previous
import functools

import jax
import jax.numpy as jnp
from jax import lax
from jax.experimental import pallas as pl
from jax.experimental.pallas import tpu as pltpu
from jax.experimental.pallas import tpu_sc as plsc

_NC = 2
_NS = 16
_NW = _NC * _NS
_CH = 80


def _tc_prep(hn, pos16, Wp1, Wp2):
    N, D = hn.shape
    FH = Wp1.shape[1]
    NB = 1000
    grid = (N // NB,)

    def body(hn_r, p_r, w1_r, w2_r, out_r):
        f32 = jnp.float32
        b16 = jnp.bfloat16
        i32 = jnp.int32
        pp1 = jnp.dot(p_r[...], w1_r[...], preferred_element_type=f32)
        pp2 = jnp.dot(p_r[...], w2_r[...], preferred_element_type=f32)
        pp = jnp.concatenate([pp1, pp2], axis=1)
        hi = lax.bitcast_convert_type(hn_r[...].astype(b16).astype(f32), i32)
        lo = lax.bitcast_convert_type(pp.astype(b16).astype(f32), i32)
        out_r[...] = hi | (jnp.right_shift(lo, 16) & jnp.int32(0xFFFF))

    return pl.pallas_call(
        body,
        grid=grid,
        in_specs=[
            pl.BlockSpec((NB, D), lambda i: (i, 0)),
            pl.BlockSpec((NB, pos16.shape[1]), lambda i: (i, 0)),
            pl.BlockSpec(Wp1.shape, lambda i: (0, 0)),
            pl.BlockSpec(Wp2.shape, lambda i: (0, 0)),
        ],
        out_specs=pl.BlockSpec((NB, D), lambda i: (i, 0)),
        out_shape=jax.ShapeDtypeStruct((N, D), jnp.int32),
        compiler_params=pltpu.CompilerParams(
            dimension_semantics=("arbitrary",)),
    )(hn, pos16, Wp1, Wp2)


def _sc_gather(table, src, dst):
    N, TD = table.shape
    E = src.shape[0]
    per_w = E // _NW
    n_ch = per_w // _CH
    mesh = plsc.VectorSubcoreMesh(core_axis_name="c", subcore_axis_name="s",
                                  num_cores=_NC, num_subcores=_NS)

    @functools.partial(
        pl.kernel,
        out_type=(
            jax.ShapeDtypeStruct((E, TD), jnp.int32),
            jax.ShapeDtypeStruct((E, TD), jnp.int32),
        ),
        mesh=mesh,
        scratch_types=[
            pltpu.VMEM((_CH,), jnp.int32),
            pltpu.VMEM((_CH,), jnp.int32),
            pltpu.VMEM((_CH, TD), jnp.int32),
            pltpu.VMEM((_CH, TD), jnp.int32),
            pltpu.SemaphoreType.DMA,
        ],
    )
    def k(tab_hbm, src_hbm, dst_hbm, gs_hbm, gd_hbm,
          sidx, didx, srows, drows, sem):
        wid = lax.axis_index("s") * _NC + lax.axis_index("c")
        base = wid * per_w

        def body(j, carry):
            off = base + j * _CH
            pltpu.sync_copy(src_hbm.at[pl.ds(off, _CH)], sidx)
            pltpu.sync_copy(dst_hbm.at[pl.ds(off, _CH)], didx)
            c1 = pltpu.async_copy(tab_hbm.at[sidx], srows, sem)
            c2 = pltpu.async_copy(tab_hbm.at[didx], drows, sem)
            c1.wait()
            c2.wait()
            pltpu.sync_copy(srows, gs_hbm.at[pl.ds(off, _CH)])
            pltpu.sync_copy(drows, gd_hbm.at[pl.ds(off, _CH)])
            return carry

        lax.fori_loop(0, n_ch, body, 0)

    return k(table, src, dst)


def _sc_scatter(henw, dst, zeros_nd):
    E, D = henw.shape
    N = zeros_nd.shape[0]
    per_w = E // _NW
    n_ch = per_w // _CH
    n_cp = 10
    rows_pt = N // n_cp
    mesh = plsc.VectorSubcoreMesh(core_axis_name="c", subcore_axis_name="s",
                                  num_cores=_NC, num_subcores=_NS)

    @functools.partial(
        pl.kernel,
        out_type=jax.ShapeDtypeStruct((_NC, N, D), jnp.float32),
        mesh=mesh,
        scratch_types=[
            pltpu.VMEM((_CH,), jnp.int32),
            pltpu.VMEM((_CH, D), jnp.float32),
            pltpu.VMEM_SHARED((N, D), jnp.float32),
        ],
    )
    def k(henw_hbm, dst_hbm, zeros_hbm, out_hbm, idx, rows, acc):
        c = lax.axis_index("c")
        s = lax.axis_index("s")
        wid = s * _NC + c

        @pl.when(s == 0)
        def _():
            pltpu.sync_copy(zeros_hbm, acc)

        plsc.subcore_barrier()
        base = wid * per_w

        def body(j, carry):
            off = base + j * _CH
            pltpu.sync_copy(dst_hbm.at[pl.ds(off, _CH)], idx)
            pltpu.sync_copy(henw_hbm.at[pl.ds(off, _CH)], rows)
            pltpu.sync_copy(rows, acc.at[idx], add=True)
            return carry

        lax.fori_loop(0, n_ch, body, 0)
        plsc.subcore_barrier()

        @pl.when(s < n_cp)
        def _():
            pltpu.sync_copy(acc.at[pl.ds(s * rows_pt, rows_pt)],
                            out_hbm.at[c, pl.ds(s * rows_pt, rows_pt)])

    return k(henw, dst, zeros_nd)


def _tc_edge(he, gsrc, gdst, fes, norm2, W1e, W1s, W1d, Wev2, Wf, Wfc2):
    E, D = he.shape
    TD = gsrc.shape[1]
    FH = D // 2
    NF = fes.shape[1]
    VD = Wev2.shape[1]
    EB = 512
    grid = (E // EB,)

    def body(he_r, gs_r, gd_r, fes_r, nrm_r,
             w1e_r, w1s_r, w1d_r, wev2_r, wf_r, wfc2_r,
             hen_r, henw_r):
        f32 = jnp.float32
        b16 = jnp.bfloat16
        i32 = jnp.int32
        hi_mask = jnp.int32(-65536)
        gs_w = gs_r[...]
        gd_w = gd_r[...]
        gs_hn = lax.bitcast_convert_type(gs_w & hi_mask, f32)
        gd_hn = lax.bitcast_convert_type(gd_w & hi_mask, f32)
        gs_pp = lax.bitcast_convert_type(jnp.left_shift(gs_w, 16), f32)
        gd_pp = lax.bitcast_convert_type(jnp.left_shift(gd_w, 16), f32)
        h1 = jnp.maximum(
            jnp.dot(he_r[...].astype(b16), w1e_r[...],
                    preferred_element_type=f32)
            + jnp.dot(gs_hn.astype(b16), w1s_r[...],
                      preferred_element_type=f32)
            + jnp.dot(gd_hn.astype(b16), w1d_r[...],
                      preferred_element_type=f32),
            0.0)
        v = jnp.dot(h1.astype(b16), wev2_r[...], preferred_element_type=f32)
        r = jnp.maximum(
            gs_pp[:, :FH] + gd_pp[:, FH:]
            + jnp.dot(fes_r[...], wf_r[...], preferred_element_type=f32),
            0.0)
        m = jnp.dot(r.astype(b16), wfc2_r[...],
                    preferred_element_type=f32).astype(b16)
        v16 = v.astype(b16)
        acc = []
        for q in range(VD):
            acc.append(v16[:, q:q + 1] * m[:, q * D:(q + 1) * D])
        while len(acc) > 1:
            acc = [a + b for a, b in zip(acc[::2], acc[1::2])]
        hen = he_r[...] + acc[0].astype(f32)
        hen_r[...] = hen
        henw_r[...] = hen * nrm_r[...]

    blk = lambda rows, cols: pl.BlockSpec((rows, cols), lambda i: (i, 0))
    wblk = lambda shape: pl.BlockSpec(shape, lambda i: (0, 0))
    return pl.pallas_call(
        body,
        grid=grid,
        in_specs=[
            blk(EB, D), blk(EB, TD), blk(EB, TD), blk(EB, NF), blk(EB, 1),
            wblk(W1e.shape), wblk(W1s.shape), wblk(W1d.shape),
            wblk(Wev2.shape), wblk(Wf.shape), wblk(Wfc2.shape),
        ],
        out_specs=[blk(EB, D), blk(EB, D)],
        out_shape=[
            jax.ShapeDtypeStruct((E, D), jnp.float32),
            jax.ShapeDtypeStruct((E, D), jnp.float32),
        ],
        compiler_params=pltpu.CompilerParams(
            dimension_semantics=("arbitrary",)),
    )(he, gsrc, gdst, fes, norm2, W1e, W1s, W1d, Wev2, Wf, Wfc2)


def _tc_node(hn, parts, Wn1a, Wn1b, Wn2):
    N, D = hn.shape
    NB = 1000
    grid = (N // NB,)

    def body(hn_r, p_r, w1a_r, w1b_r, w2_r, out_r):
        f32 = jnp.float32
        t = p_r[0] + p_r[1]
        h = jnp.maximum(
            jnp.dot(hn_r[...], w1a_r[...], preferred_element_type=f32)
            + jnp.dot(t, w1b_r[...], preferred_element_type=f32),
            0.0)
        out_r[...] = hn_r[...] + jnp.dot(h, w2_r[...],
                                         preferred_element_type=f32)

    return pl.pallas_call(
        body,
        grid=grid,
        in_specs=[
            pl.BlockSpec((NB, D), lambda i: (i, 0)),
            pl.BlockSpec((2, NB, D), lambda i: (0, i, 0)),
            pl.BlockSpec(Wn1a.shape, lambda i: (0, 0)),
            pl.BlockSpec(Wn1b.shape, lambda i: (0, 0)),
            pl.BlockSpec(Wn2.shape, lambda i: (0, 0)),
        ],
        out_specs=pl.BlockSpec((NB, D), lambda i: (i, 0)),
        out_shape=jax.ShapeDtypeStruct((N, D), jnp.float32),
        compiler_params=pltpu.CompilerParams(
            dimension_semantics=("arbitrary",)),
    )(hn, parts, Wn1a, Wn1b, Wn2)


def kernel(hn, he, pos, fes, norm, edge_index,
           W_ev1, W_ev2, W_fc1, W_fc2, W_nu1, W_nu2):
    N, D = hn.shape
    src = edge_index[0]
    dst = edge_index[1]

    PD = 16
    pos16 = jnp.pad(pos, ((0, 0), (0, PD - pos.shape[1])))
    W1e, W1s, W1d = W_ev1[:D], W_ev1[D:2 * D], W_ev1[2 * D:]
    Wp1 = jnp.pad(W_fc1[0:3], ((0, PD - 3), (0, 0)))
    Wp2 = jnp.pad(W_fc1[3:6], ((0, PD - 3), (0, 0)))
    Wf = W_fc1[6:]
    Wn1a, Wn1b = W_nu1[:D], W_nu1[D:]
    norm2 = norm[:, None]
    zeros_nd = jnp.zeros((N, D), jnp.float32)

    b16 = jnp.bfloat16
    table = _tc_prep(hn, pos16, Wp1, Wp2)
    gsrc, gdst = _sc_gather(table, src, dst)
    hen, henw = _tc_edge(he, gsrc, gdst, fes, norm2,
                         W1e.astype(b16), W1s.astype(b16), W1d.astype(b16),
                         W_ev2.astype(b16), Wf, W_fc2.astype(b16))
    parts = _sc_scatter(henw, dst, zeros_nd)
    hnn = _tc_node(hn, parts, Wn1a, Wn1b, W_nu2)
    return (hnn, hen)


# speedup vs baseline: 3.7304x; 1.0556x over previous
import functools

import jax
import jax.numpy as jnp
from jax import lax
from jax.experimental import pallas as pl
from jax.experimental.pallas import tpu as pltpu
from jax.experimental.pallas import tpu_sc as plsc

_NC = 2
_NS = 16
_NW = _NC * _NS
_CH = 80


def _tc_prep(hn, pos16, Wp1, Wp2):
    N, D = hn.shape
    FH = Wp1.shape[1]
    NB = 1000
    grid = (N // NB,)

    def body(hn_r, p_r, w1_r, w2_r, out_r):
        f32 = jnp.float32
        b16 = jnp.bfloat16
        i32 = jnp.int32
        pp1 = jnp.dot(p_r[...], w1_r[...], preferred_element_type=f32)
        pp2 = jnp.dot(p_r[...], w2_r[...], preferred_element_type=f32)
        pp = jnp.concatenate([pp1, pp2], axis=1)
        hi = lax.bitcast_convert_type(hn_r[...].astype(b16).astype(f32), i32)
        lo = lax.bitcast_convert_type(pp.astype(b16).astype(f32), i32)
        out_r[...] = hi | (jnp.right_shift(lo, 16) & jnp.int32(0xFFFF))

    return pl.pallas_call(
        body,
        grid=grid,
        in_specs=[
            pl.BlockSpec((NB, D), lambda i: (i, 0)),
            pl.BlockSpec((NB, pos16.shape[1]), lambda i: (i, 0)),
            pl.BlockSpec(Wp1.shape, lambda i: (0, 0)),
            pl.BlockSpec(Wp2.shape, lambda i: (0, 0)),
        ],
        out_specs=pl.BlockSpec((NB, D), lambda i: (i, 0)),
        out_shape=jax.ShapeDtypeStruct((N, D), jnp.int32),
        compiler_params=pltpu.CompilerParams(
            dimension_semantics=("arbitrary",)),
    )(hn, pos16, Wp1, Wp2)


def _sc_gather(table, src, dst):
    N, TD = table.shape
    E = src.shape[0]
    per_w = E // _NW
    n_ch = per_w // _CH
    mesh = plsc.VectorSubcoreMesh(core_axis_name="c", subcore_axis_name="s",
                                  num_cores=_NC, num_subcores=_NS)

    @functools.partial(
        pl.kernel,
        out_type=(
            jax.ShapeDtypeStruct((E, TD), jnp.int32),
            jax.ShapeDtypeStruct((E, TD), jnp.int32),
        ),
        mesh=mesh,
        scratch_types=[
            pltpu.VMEM((_CH,), jnp.int32),
            pltpu.VMEM((_CH,), jnp.int32),
            pltpu.VMEM((_CH, TD), jnp.int32),
            pltpu.VMEM((_CH, TD), jnp.int32),
            pltpu.SemaphoreType.DMA,
        ],
    )
    def k(tab_hbm, src_hbm, dst_hbm, gs_hbm, gd_hbm,
          sidx, didx, srows, drows, sem):
        wid = lax.axis_index("s") * _NC + lax.axis_index("c")
        base = wid * per_w

        def body(j, carry):
            off = base + j * _CH
            pltpu.sync_copy(src_hbm.at[pl.ds(off, _CH)], sidx)
            pltpu.sync_copy(dst_hbm.at[pl.ds(off, _CH)], didx)
            c1 = pltpu.async_copy(tab_hbm.at[sidx], srows, sem)
            c2 = pltpu.async_copy(tab_hbm.at[didx], drows, sem)
            c1.wait()
            c2.wait()
            pltpu.sync_copy(srows, gs_hbm.at[pl.ds(off, _CH)])
            pltpu.sync_copy(drows, gd_hbm.at[pl.ds(off, _CH)])
            return carry

        lax.fori_loop(0, n_ch, body, 0)

    return k(table, src, dst)


def _sc_scatter(henw, dst, zeros_nd):
    E, D = henw.shape
    N = zeros_nd.shape[0]
    per_w = E // _NW
    n_ch = per_w // _CH
    n_cp = 10
    rows_pt = N // n_cp
    mesh = plsc.VectorSubcoreMesh(core_axis_name="c", subcore_axis_name="s",
                                  num_cores=_NC, num_subcores=_NS)

    @functools.partial(
        pl.kernel,
        out_type=jax.ShapeDtypeStruct((_NC, N, D), jnp.float32),
        mesh=mesh,
        scratch_types=[
            pltpu.VMEM((_CH,), jnp.int32),
            pltpu.VMEM((_CH, D), jnp.float32),
            pltpu.VMEM_SHARED((N, D), jnp.float32),
        ],
    )
    def k(henw_hbm, dst_hbm, zeros_hbm, out_hbm, idx, rows, acc):
        c = lax.axis_index("c")
        s = lax.axis_index("s")
        wid = s * _NC + c

        @pl.when(s == 0)
        def _():
            pltpu.sync_copy(zeros_hbm, acc)

        plsc.subcore_barrier()
        base = wid * per_w

        def body(j, carry):
            off = base + j * _CH
            pltpu.sync_copy(dst_hbm.at[pl.ds(off, _CH)], idx)
            pltpu.sync_copy(henw_hbm.at[pl.ds(off, _CH)], rows)
            pltpu.sync_copy(rows, acc.at[idx], add=True)
            return carry

        lax.fori_loop(0, n_ch, body, 0)
        plsc.subcore_barrier()

        @pl.when(s < n_cp)
        def _():
            pltpu.sync_copy(acc.at[pl.ds(s * rows_pt, rows_pt)],
                            out_hbm.at[c, pl.ds(s * rows_pt, rows_pt)])

    return k(henw, dst, zeros_nd)


def _tc_edge(he, gsrc, gdst, fes, norm2, W1e, W1s, W1d, Wev2, Wf, Wfc2):
    E, D = he.shape
    TD = gsrc.shape[1]
    FH = D // 2
    NF = fes.shape[1]
    VD = Wev2.shape[1]
    EB = 1024
    grid = (E // EB,)

    def body(he_r, gs_r, gd_r, fes_r, nrm_r,
             w1e_r, w1s_r, w1d_r, wev2_r, wf_r, wfc2_r,
             hen_r, henw_r):
        f32 = jnp.float32
        b16 = jnp.bfloat16
        i32 = jnp.int32
        hi_mask = jnp.int32(-65536)
        gs_w = gs_r[...]
        gd_w = gd_r[...]
        gs_hn = lax.bitcast_convert_type(gs_w & hi_mask, f32)
        gd_hn = lax.bitcast_convert_type(gd_w & hi_mask, f32)
        gs_pp = lax.bitcast_convert_type(jnp.left_shift(gs_w, 16), f32)
        gd_pp = lax.bitcast_convert_type(jnp.left_shift(gd_w, 16), f32)
        h1 = jnp.maximum(
            jnp.dot(he_r[...].astype(b16), w1e_r[...],
                    preferred_element_type=f32)
            + jnp.dot(gs_hn.astype(b16), w1s_r[...],
                      preferred_element_type=f32)
            + jnp.dot(gd_hn.astype(b16), w1d_r[...],
                      preferred_element_type=f32),
            0.0)
        v = jnp.dot(h1.astype(b16), wev2_r[...], preferred_element_type=f32)
        r = jnp.maximum(
            gs_pp[:, :FH] + gd_pp[:, FH:]
            + jnp.dot(fes_r[...], wf_r[...], preferred_element_type=f32),
            0.0)
        m = jnp.dot(r.astype(b16), wfc2_r[...],
                    preferred_element_type=f32).astype(b16)
        v16 = v.astype(b16)
        acc = []
        for q in range(VD):
            acc.append(v16[:, q:q + 1] * m[:, q * D:(q + 1) * D])
        while len(acc) > 1:
            acc = [a + b for a, b in zip(acc[::2], acc[1::2])]
        hen = he_r[...] + acc[0].astype(f32)
        hen_r[...] = hen
        henw_r[...] = hen * nrm_r[...]

    blk = lambda rows, cols: pl.BlockSpec((rows, cols), lambda i: (i, 0))
    wblk = lambda shape: pl.BlockSpec(shape, lambda i: (0, 0))
    return pl.pallas_call(
        body,
        grid=grid,
        in_specs=[
            blk(EB, D), blk(EB, TD), blk(EB, TD), blk(EB, NF), blk(EB, 1),
            wblk(W1e.shape), wblk(W1s.shape), wblk(W1d.shape),
            wblk(Wev2.shape), wblk(Wf.shape), wblk(Wfc2.shape),
        ],
        out_specs=[blk(EB, D), blk(EB, D)],
        out_shape=[
            jax.ShapeDtypeStruct((E, D), jnp.float32),
            jax.ShapeDtypeStruct((E, D), jnp.float32),
        ],
        compiler_params=pltpu.CompilerParams(
            dimension_semantics=("arbitrary",)),
    )(he, gsrc, gdst, fes, norm2, W1e, W1s, W1d, Wev2, Wf, Wfc2)


def _tc_node(hn, parts, Wn1a, Wn1b, Wn2):
    N, D = hn.shape
    NB = 1000
    grid = (N // NB,)

    def body(hn_r, p_r, w1a_r, w1b_r, w2_r, out_r):
        f32 = jnp.float32
        t = p_r[0] + p_r[1]
        h = jnp.maximum(
            jnp.dot(hn_r[...], w1a_r[...], preferred_element_type=f32)
            + jnp.dot(t, w1b_r[...], preferred_element_type=f32),
            0.0)
        out_r[...] = hn_r[...] + jnp.dot(h, w2_r[...],
                                         preferred_element_type=f32)

    return pl.pallas_call(
        body,
        grid=grid,
        in_specs=[
            pl.BlockSpec((NB, D), lambda i: (i, 0)),
            pl.BlockSpec((2, NB, D), lambda i: (0, i, 0)),
            pl.BlockSpec(Wn1a.shape, lambda i: (0, 0)),
            pl.BlockSpec(Wn1b.shape, lambda i: (0, 0)),
            pl.BlockSpec(Wn2.shape, lambda i: (0, 0)),
        ],
        out_specs=pl.BlockSpec((NB, D), lambda i: (i, 0)),
        out_shape=jax.ShapeDtypeStruct((N, D), jnp.float32),
        compiler_params=pltpu.CompilerParams(
            dimension_semantics=("arbitrary",)),
    )(hn, parts, Wn1a, Wn1b, Wn2)


def kernel(hn, he, pos, fes, norm, edge_index,
           W_ev1, W_ev2, W_fc1, W_fc2, W_nu1, W_nu2):
    N, D = hn.shape
    src = edge_index[0]
    dst = edge_index[1]

    PD = 16
    pos16 = jnp.pad(pos, ((0, 0), (0, PD - pos.shape[1])))
    W1e, W1s, W1d = W_ev1[:D], W_ev1[D:2 * D], W_ev1[2 * D:]
    Wp1 = jnp.pad(W_fc1[0:3], ((0, PD - 3), (0, 0)))
    Wp2 = jnp.pad(W_fc1[3:6], ((0, PD - 3), (0, 0)))
    Wf = W_fc1[6:]
    Wn1a, Wn1b = W_nu1[:D], W_nu1[D:]
    norm2 = norm[:, None]
    zeros_nd = jnp.zeros((N, D), jnp.float32)

    b16 = jnp.bfloat16
    table = _tc_prep(hn, pos16, Wp1, Wp2)
    gsrc, gdst = _sc_gather(table, src, dst)
    hen, henw = _tc_edge(he, gsrc, gdst, fes, norm2,
                         W1e.astype(b16), W1s.astype(b16), W1d.astype(b16),
                         W_ev2.astype(b16), Wf, W_fc2.astype(b16))
    parts = _sc_scatter(henw, dst, zeros_nd)
    hnn = _tc_node(hn, parts, Wn1a, Wn1b, W_nu2)
    return (hnn, hen)


# speedup vs baseline: 3.8337x; 1.0277x over previous
import functools

import jax
import jax.numpy as jnp
from jax import lax
from jax.experimental import pallas as pl
from jax.experimental.pallas import tpu as pltpu
from jax.experimental.pallas import tpu_sc as plsc

_NC = 2
_NS = 16
_NW = _NC * _NS
_CH = 80


def _tc_prep(hn, pos16, Wp1, Wp2):
    N, D = hn.shape
    FH = Wp1.shape[1]
    NB = 1000
    grid = (N // NB,)

    def body(hn_r, p_r, w1_r, w2_r, out_r):
        f32 = jnp.float32
        b16 = jnp.bfloat16
        i32 = jnp.int32
        pp1 = jnp.dot(p_r[...], w1_r[...], preferred_element_type=f32)
        pp2 = jnp.dot(p_r[...], w2_r[...], preferred_element_type=f32)
        pp = jnp.concatenate([pp1, pp2], axis=1)
        hi = lax.bitcast_convert_type(hn_r[...].astype(b16).astype(f32), i32)
        lo = lax.bitcast_convert_type(pp.astype(b16).astype(f32), i32)
        out_r[...] = hi | (jnp.right_shift(lo, 16) & jnp.int32(0xFFFF))

    return pl.pallas_call(
        body,
        grid=grid,
        in_specs=[
            pl.BlockSpec((NB, D), lambda i: (i, 0)),
            pl.BlockSpec((NB, pos16.shape[1]), lambda i: (i, 0)),
            pl.BlockSpec(Wp1.shape, lambda i: (0, 0)),
            pl.BlockSpec(Wp2.shape, lambda i: (0, 0)),
        ],
        out_specs=pl.BlockSpec((NB, D), lambda i: (i, 0)),
        out_shape=jax.ShapeDtypeStruct((N, D), jnp.int32),
        compiler_params=pltpu.CompilerParams(
            dimension_semantics=("arbitrary",)),
    )(hn, pos16, Wp1, Wp2)


def _sc_gather(table, src, dst):
    N, TD = table.shape
    E = src.shape[0]
    per_w = E // _NW
    n_ch = per_w // _CH
    mesh = plsc.VectorSubcoreMesh(core_axis_name="c", subcore_axis_name="s",
                                  num_cores=_NC, num_subcores=_NS)

    @functools.partial(
        pl.kernel,
        out_type=(
            jax.ShapeDtypeStruct((E, TD), jnp.int32),
            jax.ShapeDtypeStruct((E, TD), jnp.int32),
        ),
        mesh=mesh,
        scratch_types=[
            pltpu.VMEM((_CH,), jnp.int32),
            pltpu.VMEM((_CH,), jnp.int32),
            pltpu.VMEM((_CH, TD), jnp.int32),
            pltpu.VMEM((_CH, TD), jnp.int32),
            pltpu.SemaphoreType.DMA,
        ],
    )
    def k(tab_hbm, src_hbm, dst_hbm, gs_hbm, gd_hbm,
          sidx, didx, srows, drows, sem):
        wid = lax.axis_index("s") * _NC + lax.axis_index("c")
        base = wid * per_w

        def body(j, carry):
            off = base + j * _CH
            pltpu.sync_copy(src_hbm.at[pl.ds(off, _CH)], sidx)
            pltpu.sync_copy(dst_hbm.at[pl.ds(off, _CH)], didx)
            c1 = pltpu.async_copy(tab_hbm.at[sidx], srows, sem)
            c2 = pltpu.async_copy(tab_hbm.at[didx], drows, sem)
            c1.wait()
            c2.wait()
            pltpu.sync_copy(srows, gs_hbm.at[pl.ds(off, _CH)])
            pltpu.sync_copy(drows, gd_hbm.at[pl.ds(off, _CH)])
            return carry

        lax.fori_loop(0, n_ch, body, 0)

    return k(table, src, dst)


def _sc_scatter(henw, dst, zeros_nd):
    E, D = henw.shape
    N = zeros_nd.shape[0]
    per_w = E // _NW
    n_ch = per_w // _CH
    n_cp = 10
    rows_pt = N // n_cp
    mesh = plsc.VectorSubcoreMesh(core_axis_name="c", subcore_axis_name="s",
                                  num_cores=_NC, num_subcores=_NS)

    @functools.partial(
        pl.kernel,
        out_type=jax.ShapeDtypeStruct((_NC, N, D), jnp.float32),
        mesh=mesh,
        scratch_types=[
            pltpu.VMEM((_CH,), jnp.int32),
            pltpu.VMEM((_CH, D), jnp.float32),
            pltpu.VMEM_SHARED((N, D), jnp.float32),
        ],
    )
    def k(henw_hbm, dst_hbm, zeros_hbm, out_hbm, idx, rows, acc):
        c = lax.axis_index("c")
        s = lax.axis_index("s")
        wid = s * _NC + c

        @pl.when(s == 0)
        def _():
            pltpu.sync_copy(zeros_hbm, acc)

        plsc.subcore_barrier()
        base = wid * per_w

        def body(j, carry):
            off = base + j * _CH
            pltpu.sync_copy(dst_hbm.at[pl.ds(off, _CH)], idx)
            pltpu.sync_copy(henw_hbm.at[pl.ds(off, _CH)], rows)
            pltpu.sync_copy(rows, acc.at[idx], add=True)
            return carry

        lax.fori_loop(0, n_ch, body, 0)
        plsc.subcore_barrier()

        @pl.when(s < n_cp)
        def _():
            pltpu.sync_copy(acc.at[pl.ds(s * rows_pt, rows_pt)],
                            out_hbm.at[c, pl.ds(s * rows_pt, rows_pt)])

    return k(henw, dst, zeros_nd)


def _tc_edge(he, gsrc, gdst, fes, norm2, W1e, W1s, W1d, Wev2, Wf, Wfc2):
    E, D = he.shape
    TD = gsrc.shape[1]
    FH = D // 2
    NF = fes.shape[1]
    VD = Wev2.shape[1]
    EB = 2048
    grid = (E // EB,)

    def body(he_r, gs_r, gd_r, fes_r, nrm_r,
             w1e_r, w1s_r, w1d_r, wev2_r, wf_r, wfc2_r,
             hen_r, henw_r):
        f32 = jnp.float32
        b16 = jnp.bfloat16
        i32 = jnp.int32
        hi_mask = jnp.int32(-65536)
        gs_w = gs_r[...]
        gd_w = gd_r[...]
        gs_hn = lax.bitcast_convert_type(gs_w & hi_mask, f32)
        gd_hn = lax.bitcast_convert_type(gd_w & hi_mask, f32)
        gs_pp = lax.bitcast_convert_type(jnp.left_shift(gs_w, 16), f32)
        gd_pp = lax.bitcast_convert_type(jnp.left_shift(gd_w, 16), f32)
        h1 = jnp.maximum(
            jnp.dot(he_r[...].astype(b16), w1e_r[...],
                    preferred_element_type=f32)
            + jnp.dot(gs_hn.astype(b16), w1s_r[...],
                      preferred_element_type=f32)
            + jnp.dot(gd_hn.astype(b16), w1d_r[...],
                      preferred_element_type=f32),
            0.0)
        v = jnp.dot(h1.astype(b16), wev2_r[...], preferred_element_type=f32)
        r = jnp.maximum(
            gs_pp[:, :FH] + gd_pp[:, FH:]
            + jnp.dot(fes_r[...], wf_r[...], preferred_element_type=f32),
            0.0)
        m = jnp.dot(r.astype(b16), wfc2_r[...],
                    preferred_element_type=f32).astype(b16)
        v16 = v.astype(b16)
        acc = []
        for q in range(VD):
            acc.append(v16[:, q:q + 1] * m[:, q * D:(q + 1) * D])
        while len(acc) > 1:
            acc = [a + b for a, b in zip(acc[::2], acc[1::2])]
        hen = he_r[...] + acc[0].astype(f32)
        hen_r[...] = hen
        henw_r[...] = hen * nrm_r[...]

    blk = lambda rows, cols: pl.BlockSpec((rows, cols), lambda i: (i, 0))
    wblk = lambda shape: pl.BlockSpec(shape, lambda i: (0, 0))
    return pl.pallas_call(
        body,
        grid=grid,
        in_specs=[
            blk(EB, D), blk(EB, TD), blk(EB, TD), blk(EB, NF), blk(EB, 1),
            wblk(W1e.shape), wblk(W1s.shape), wblk(W1d.shape),
            wblk(Wev2.shape), wblk(Wf.shape), wblk(Wfc2.shape),
        ],
        out_specs=[blk(EB, D), blk(EB, D)],
        out_shape=[
            jax.ShapeDtypeStruct((E, D), jnp.float32),
            jax.ShapeDtypeStruct((E, D), jnp.float32),
        ],
        compiler_params=pltpu.CompilerParams(
            dimension_semantics=("arbitrary",)),
    )(he, gsrc, gdst, fes, norm2, W1e, W1s, W1d, Wev2, Wf, Wfc2)


def _tc_node(hn, parts, Wn1a, Wn1b, Wn2):
    N, D = hn.shape
    NB = 1000
    grid = (N // NB,)

    def body(hn_r, p_r, w1a_r, w1b_r, w2_r, out_r):
        f32 = jnp.float32
        t = p_r[0] + p_r[1]
        h = jnp.maximum(
            jnp.dot(hn_r[...], w1a_r[...], preferred_element_type=f32)
            + jnp.dot(t, w1b_r[...], preferred_element_type=f32),
            0.0)
        out_r[...] = hn_r[...] + jnp.dot(h, w2_r[...],
                                         preferred_element_type=f32)

    return pl.pallas_call(
        body,
        grid=grid,
        in_specs=[
            pl.BlockSpec((NB, D), lambda i: (i, 0)),
            pl.BlockSpec((2, NB, D), lambda i: (0, i, 0)),
            pl.BlockSpec(Wn1a.shape, lambda i: (0, 0)),
            pl.BlockSpec(Wn1b.shape, lambda i: (0, 0)),
            pl.BlockSpec(Wn2.shape, lambda i: (0, 0)),
        ],
        out_specs=pl.BlockSpec((NB, D), lambda i: (i, 0)),
        out_shape=jax.ShapeDtypeStruct((N, D), jnp.float32),
        compiler_params=pltpu.CompilerParams(
            dimension_semantics=("arbitrary",)),
    )(hn, parts, Wn1a, Wn1b, Wn2)


def kernel(hn, he, pos, fes, norm, edge_index,
           W_ev1, W_ev2, W_fc1, W_fc2, W_nu1, W_nu2):
    N, D = hn.shape
    src = edge_index[0]
    dst = edge_index[1]

    PD = 16
    pos16 = jnp.pad(pos, ((0, 0), (0, PD - pos.shape[1])))
    W1e, W1s, W1d = W_ev1[:D], W_ev1[D:2 * D], W_ev1[2 * D:]
    Wp1 = jnp.pad(W_fc1[0:3], ((0, PD - 3), (0, 0)))
    Wp2 = jnp.pad(W_fc1[3:6], ((0, PD - 3), (0, 0)))
    Wf = W_fc1[6:]
    Wn1a, Wn1b = W_nu1[:D], W_nu1[D:]
    norm2 = norm[:, None]
    zeros_nd = jnp.zeros((N, D), jnp.float32)

    b16 = jnp.bfloat16
    table = _tc_prep(hn, pos16, Wp1, Wp2)
    gsrc, gdst = _sc_gather(table, src, dst)
    hen, henw = _tc_edge(he, gsrc, gdst, fes, norm2,
                         W1e.astype(b16), W1s.astype(b16), W1d.astype(b16),
                         W_ev2.astype(b16), Wf, W_fc2.astype(b16))
    parts = _sc_scatter(henw, dst, zeros_nd)
    hnn = _tc_node(hn, parts, Wn1a, Wn1b, W_nu2)
    return (hnn, hen)


# speedup vs baseline: 5.0352x; 1.3134x over previous
import functools

import jax
import jax.numpy as jnp
from jax import lax
from jax.experimental import pallas as pl
from jax.experimental.pallas import tpu as pltpu
from jax.experimental.pallas import tpu_sc as plsc

_NC = 2
_NS = 16
_NW = _NC * _NS
_CH = 80
_C = 5
_EB = 2000


def _tc_prep(hn, pos16, Wp1, Wp2):
    N, D = hn.shape
    NB = 1000
    grid = (N // NB,)

    def body(hn_r, p_r, w1_r, w2_r, out_r):
        f32 = jnp.float32
        b16 = jnp.bfloat16
        i32 = jnp.int32
        pp1 = jnp.dot(p_r[...], w1_r[...], preferred_element_type=f32)
        pp2 = jnp.dot(p_r[...], w2_r[...], preferred_element_type=f32)
        pp = jnp.concatenate([pp1, pp2], axis=1)
        hi = lax.bitcast_convert_type(hn_r[...].astype(b16).astype(f32), i32)
        lo = lax.bitcast_convert_type(pp.astype(b16).astype(f32), i32)
        out_r[...] = hi | (jnp.right_shift(lo, 16) & jnp.int32(0xFFFF))

    return pl.pallas_call(
        body,
        grid=grid,
        in_specs=[
            pl.BlockSpec((NB, D), lambda i: (i, 0)),
            pl.BlockSpec((NB, pos16.shape[1]), lambda i: (i, 0)),
            pl.BlockSpec(Wp1.shape, lambda i: (0, 0)),
            pl.BlockSpec(Wp2.shape, lambda i: (0, 0)),
        ],
        out_specs=pl.BlockSpec((NB, D), lambda i: (i, 0)),
        out_shape=jax.ShapeDtypeStruct((N, D), jnp.int32),
        compiler_params=pltpu.CompilerParams(
            dimension_semantics=("arbitrary",)),
    )(hn, pos16, Wp1, Wp2)


def _sc_gather(table, src, dst, off, ec):
    N, TD = table.shape
    per_w = ec // _NW
    n_ch = per_w // _CH
    mesh = plsc.VectorSubcoreMesh(core_axis_name="c", subcore_axis_name="s",
                                  num_cores=_NC, num_subcores=_NS)

    @functools.partial(
        pl.kernel,
        out_type=(
            jax.ShapeDtypeStruct((ec, TD), jnp.int32),
            jax.ShapeDtypeStruct((ec, TD), jnp.int32),
        ),
        mesh=mesh,
        scratch_types=[
            pltpu.VMEM((_CH,), jnp.int32),
            pltpu.VMEM((_CH,), jnp.int32),
            pltpu.VMEM((_CH, TD), jnp.int32),
            pltpu.VMEM((_CH, TD), jnp.int32),
            pltpu.SemaphoreType.DMA,
        ],
    )
    def k(tab_hbm, src_hbm, dst_hbm, gs_hbm, gd_hbm,
          sidx, didx, srows, drows, sem):
        wid = lax.axis_index("s") * _NC + lax.axis_index("c")
        base = wid * per_w

        def body(j, carry):
            loc = base + j * _CH
            pltpu.sync_copy(src_hbm.at[pl.ds(off + loc, _CH)], sidx)
            pltpu.sync_copy(dst_hbm.at[pl.ds(off + loc, _CH)], didx)
            c1 = pltpu.async_copy(tab_hbm.at[sidx], srows, sem)
            c2 = pltpu.async_copy(tab_hbm.at[didx], drows, sem)
            c1.wait()
            c2.wait()
            pltpu.sync_copy(srows, gs_hbm.at[pl.ds(loc, _CH)])
            pltpu.sync_copy(drows, gd_hbm.at[pl.ds(loc, _CH)])
            return carry

        lax.fori_loop(0, n_ch, body, 0)

    return k(table, src, dst)


def _sc_scatter(henw, dst, off, zeros_nd):
    ec, D = henw.shape
    N = zeros_nd.shape[0]
    per_w = ec // _NW
    n_ch = per_w // _CH
    n_cp = 10
    rows_pt = N // n_cp
    mesh = plsc.VectorSubcoreMesh(core_axis_name="c", subcore_axis_name="s",
                                  num_cores=_NC, num_subcores=_NS)

    @functools.partial(
        pl.kernel,
        out_type=jax.ShapeDtypeStruct((_NC, N, D), jnp.float32),
        mesh=mesh,
        scratch_types=[
            pltpu.VMEM((_CH,), jnp.int32),
            pltpu.VMEM((_CH, D), jnp.float32),
            pltpu.VMEM_SHARED((N, D), jnp.float32),
        ],
    )
    def k(henw_hbm, dst_hbm, zeros_hbm, out_hbm, idx, rows, acc):
        c = lax.axis_index("c")
        s = lax.axis_index("s")
        wid = s * _NC + c

        @pl.when(s == 0)
        def _():
            pltpu.sync_copy(zeros_hbm, acc)

        plsc.subcore_barrier()
        base = wid * per_w

        def body(j, carry):
            loc = base + j * _CH
            pltpu.sync_copy(dst_hbm.at[pl.ds(off + loc, _CH)], idx)
            pltpu.sync_copy(henw_hbm.at[pl.ds(loc, _CH)], rows)
            pltpu.sync_copy(rows, acc.at[idx], add=True)
            return carry

        lax.fori_loop(0, n_ch, body, 0)
        plsc.subcore_barrier()

        @pl.when(s < n_cp)
        def _():
            pltpu.sync_copy(acc.at[pl.ds(s * rows_pt, rows_pt)],
                            out_hbm.at[c, pl.ds(s * rows_pt, rows_pt)])

    return k(henw, dst, zeros_nd)


def _tc_edge(hen_buf, he, gsrc, gdst, fes, norm2,
             W1e, W1s, W1d, Wev2, Wf, Wfc2, k0):
    E, D = he.shape
    ec, TD = gsrc.shape
    FH = D // 2
    NF = fes.shape[1]
    VD = Wev2.shape[1]
    grid = (ec // _EB,)

    def body(*refs):
        if hen_buf is None:
            (he_r, gs_r, gd_r, fes_r, nrm_r,
             w1e_r, w1s_r, w1d_r, wev2_r, wf_r, wfc2_r,
             hen_r, henw_r) = refs
        else:
            (_, he_r, gs_r, gd_r, fes_r, nrm_r,
             w1e_r, w1s_r, w1d_r, wev2_r, wf_r, wfc2_r,
             hen_r, henw_r) = refs
        f32 = jnp.float32
        b16 = jnp.bfloat16
        hi_mask = jnp.int32(-65536)
        gs_w = gs_r[...]
        gd_w = gd_r[...]
        gs_hn = lax.bitcast_convert_type(gs_w & hi_mask, f32)
        gd_hn = lax.bitcast_convert_type(gd_w & hi_mask, f32)
        gs_pp = lax.bitcast_convert_type(jnp.left_shift(gs_w, 16), f32)
        gd_pp = lax.bitcast_convert_type(jnp.left_shift(gd_w, 16), f32)
        h1 = jnp.maximum(
            jnp.dot(he_r[...].astype(b16), w1e_r[...],
                    preferred_element_type=f32)
            + jnp.dot(gs_hn.astype(b16), w1s_r[...],
                      preferred_element_type=f32)
            + jnp.dot(gd_hn.astype(b16), w1d_r[...],
                      preferred_element_type=f32),
            0.0)
        v = jnp.dot(h1.astype(b16), wev2_r[...], preferred_element_type=f32)
        r = jnp.maximum(
            gs_pp[:, :FH] + gd_pp[:, FH:]
            + jnp.dot(fes_r[...], wf_r[...], preferred_element_type=f32),
            0.0)
        m = jnp.dot(r.astype(b16), wfc2_r[...],
                    preferred_element_type=f32).astype(b16)
        v16 = v.astype(b16)
        acc = []
        for q in range(VD):
            acc.append(v16[:, q:q + 1] * m[:, q * D:(q + 1) * D])
        while len(acc) > 1:
            acc = [a + b for a, b in zip(acc[::2], acc[1::2])]
        hen = he_r[...] + acc[0].astype(f32)
        hen_r[...] = hen
        henw_r[...] = hen * nrm_r[...]

    cblk = lambda rows, cols: pl.BlockSpec((rows, cols),
                                           lambda i: (i + k0, 0))
    lblk = lambda rows, cols: pl.BlockSpec((rows, cols), lambda i: (i, 0))
    wblk = lambda shape: pl.BlockSpec(shape, lambda i: (0, 0))
    main_specs = [
        cblk(_EB, D), lblk(_EB, TD), lblk(_EB, TD),
        cblk(_EB, NF), cblk(_EB, 1),
        wblk(W1e.shape), wblk(W1s.shape), wblk(W1d.shape),
        wblk(Wev2.shape), wblk(Wf.shape), wblk(Wfc2.shape),
    ]
    main_args = (he, gsrc, gdst, fes, norm2,
                 W1e, W1s, W1d, Wev2, Wf, Wfc2)
    if hen_buf is None:
        in_specs = main_specs
        args = main_args
        aliases = {}
    else:
        in_specs = [pl.BlockSpec((8, D), lambda i: (0, 0))] + main_specs
        args = (hen_buf,) + main_args
        aliases = {0: 0}
    return pl.pallas_call(
        body,
        grid=grid,
        in_specs=in_specs,
        out_specs=[cblk(_EB, D), lblk(_EB, D)],
        out_shape=[
            jax.ShapeDtypeStruct((E, D), jnp.float32),
            jax.ShapeDtypeStruct((ec, D), jnp.float32),
        ],
        input_output_aliases=aliases,
        compiler_params=pltpu.CompilerParams(
            dimension_semantics=("arbitrary",)),
    )(*args)


def _tc_node(hn, parts_list, Wn1a, Wn1b, Wn2):
    N, D = hn.shape
    NB = 1000
    grid = (N // NB,)

    def body(hn_r, *rest):
        f32 = jnp.float32
        part_refs = rest[:len(parts_list)]
        w1a_r, w1b_r, w2_r, out_r = rest[len(parts_list):]
        t = part_refs[0][0] + part_refs[0][1]
        for p_r in part_refs[1:]:
            t = t + p_r[0] + p_r[1]
        h = jnp.maximum(
            jnp.dot(hn_r[...], w1a_r[...], preferred_element_type=f32)
            + jnp.dot(t, w1b_r[...], preferred_element_type=f32),
            0.0)
        out_r[...] = hn_r[...] + jnp.dot(h, w2_r[...],
                                         preferred_element_type=f32)

    return pl.pallas_call(
        body,
        grid=grid,
        in_specs=[pl.BlockSpec((NB, D), lambda i: (i, 0))]
        + [pl.BlockSpec((2, NB, D), lambda i: (0, i, 0))
           for _ in parts_list]
        + [
            pl.BlockSpec(Wn1a.shape, lambda i: (0, 0)),
            pl.BlockSpec(Wn1b.shape, lambda i: (0, 0)),
            pl.BlockSpec(Wn2.shape, lambda i: (0, 0)),
        ],
        out_specs=pl.BlockSpec((NB, D), lambda i: (i, 0)),
        out_shape=jax.ShapeDtypeStruct((N, D), jnp.float32),
        compiler_params=pltpu.CompilerParams(
            dimension_semantics=("arbitrary",)),
    )(hn, *parts_list, Wn1a, Wn1b, Wn2)


def kernel(hn, he, pos, fes, norm, edge_index,
           W_ev1, W_ev2, W_fc1, W_fc2, W_nu1, W_nu2):
    N, D = hn.shape
    E = he.shape[0]
    src = edge_index[0]
    dst = edge_index[1]

    PD = 16
    pos16 = jnp.pad(pos, ((0, 0), (0, PD - pos.shape[1])))
    W1e, W1s, W1d = W_ev1[:D], W_ev1[D:2 * D], W_ev1[2 * D:]
    Wp1 = jnp.pad(W_fc1[0:3], ((0, PD - 3), (0, 0)))
    Wp2 = jnp.pad(W_fc1[3:6], ((0, PD - 3), (0, 0)))
    Wf = W_fc1[6:]
    Wn1a, Wn1b = W_nu1[:D], W_nu1[D:]
    norm2 = norm[:, None]
    zeros_nd = jnp.zeros((N, D), jnp.float32)

    b16 = jnp.bfloat16
    W1e = W1e.astype(b16)
    W1s = W1s.astype(b16)
    W1d = W1d.astype(b16)
    Wev2 = W_ev2.astype(b16)
    Wfc2 = W_fc2.astype(b16)

    table = _tc_prep(hn, pos16, Wp1, Wp2)

    ec = E // _C
    gathered = [_sc_gather(table, src, dst, k * ec, ec) for k in range(_C)]

    hen_buf = None
    parts_list = []
    for k in range(_C):
        gsrc, gdst = gathered[k]
        hen_buf, henw = _tc_edge(hen_buf, he, gsrc, gdst, fes, norm2,
                                 W1e, W1s, W1d, Wev2, Wf, Wfc2,
                                 k * (ec // _EB))
        parts_list.append(_sc_scatter(henw, dst, k * ec, zeros_nd))

    hnn = _tc_node(hn, parts_list, Wn1a, Wn1b, W_nu2)
    return (hnn, hen_buf)
